# baseline (speedup 1.0000x reference)
import functools

import jax
import jax.numpy as jnp
from jax import lax
from jax.experimental import pallas as pl
from jax.experimental.pallas import tpu as pltpu
from jax.experimental.pallas import tpu_sc as plsc

B = 16384
L = 200
D = 64
NUM_CORES = 2
NUM_SUBCORES = 16
NUM_WORKERS = NUM_CORES * NUM_SUBCORES
BW = B // NUM_WORKERS
FW = BW // 2
LPS = 20
NUM_SUPERS = L // LPS

_mesh = plsc.VectorSubcoreMesh(core_axis_name="c", subcore_axis_name="s")


@functools.partial(
    pl.kernel,
    mesh=_mesh,
    out_type=jax.ShapeDtypeStruct((L, B // 2, 2 * D), jnp.float32),
    scratch_types=[
        pltpu.VMEM((LPS, 2, FW), jnp.int32),
        pltpu.VMEM((LPS, 2, FW), jnp.int32),
        pltpu.VMEM((2, FW, D), jnp.float32),
        pltpu.VMEM((2, FW, D), jnp.float32),
        pltpu.SemaphoreType.DMA,
        pltpu.SemaphoreType.DMA,
        pltpu.SemaphoreType.DMA,
        pltpu.SemaphoreType.DMA,
    ],
    compiler_params=pltpu.CompilerParams(use_tc_tiling_on_sc=False,
                                         needs_layout_passes=False),
)
def _emb_lookup_fused(idx_hbm, table_hbm, out_hbm,
                      idx_v0, idx_v1, rows0, rows1, sg0, sg1, ss0, ss1):
    wid = lax.axis_index("s") * NUM_CORES + lax.axis_index("c")
    wf = wid * FW
    idxb = (idx_v0, idx_v1)
    rows = (rows0, rows1)
    sg = (sg0, sg1)
    ss = (ss0, ss1)

    def gather_start(sb, li_local, b):
        for eo in range(2):
            pltpu.async_copy(
                table_hbm.at[idxb[sb].at[li_local, eo]],
                rows[b].at[eo], sg[b])

    def gather_wait(b):
        for eo in range(2):
            pltpu.make_async_copy(
                table_hbm.at[idx_v0.at[0, 0]], rows[b].at[eo], sg[b]).wait()

    def store_start(l, b):
        for eo in range(2):
            pltpu.async_copy(
                rows[b].at[eo],
                out_hbm.at[l, pl.ds(wf, FW), pl.ds(eo * D, D)], ss[b])

    def store_wait(b):
        for eo in range(2):
            pltpu.make_async_copy(
                rows[b].at[eo],
                out_hbm.at[0, pl.ds(wf, FW), pl.ds(eo * D, D)], ss[b]).wait()

    pltpu.sync_copy(
        idx_hbm.at[pl.ds(0, LPS), :, pl.ds(wf, FW)], idx_v0)
    gather_start(0, 0, 1)
    store_start(0, 0)

    def super_pair(sp, _):
        for sb in (0, 1):
            s = 2 * sp + sb
            pltpu.sync_copy(
                idx_hbm.at[pl.ds(s * LPS, LPS), :, pl.ds(wf, FW)], idxb[sb])

            def pos_pair(p, _):
                for b in (0, 1):
                    li = 2 * p + b
                    l = s * LPS + li
                    store_wait(b)
                    gather_start(sb, li, b)
                    gather_wait(1 - b)
                    store_start(jnp.maximum(l - 1, 0), 1 - b)
                return 0

            lax.fori_loop(0, LPS // 2, pos_pair, 0)
        return 0

    lax.fori_loop(0, NUM_SUPERS // 2, super_pair, 0)

    gather_wait(1)
    store_start(L - 1, 1)
    store_wait(0)
    store_wait(1)


def kernel(indices, table):
    idx_eo = indices.T.reshape(L, B // 2, 2).transpose(0, 2, 1)
    out_f = _emb_lookup_fused(idx_eo, table)
    return out_f.reshape(L, B, D).transpose(1, 0, 2)


# speedup vs baseline: 1.0014x; 1.0014x over previous
import functools

import jax
import jax.numpy as jnp
from jax import lax
from jax.experimental import pallas as pl
from jax.experimental.pallas import tpu as pltpu
from jax.experimental.pallas import tpu_sc as plsc

B = 16384
L = 200
D = 64
NUM_CORES = 2
NUM_SUBCORES = 16
NUM_WORKERS = NUM_CORES * NUM_SUBCORES
BW = B // NUM_WORKERS
FW = BW // 2
LPS = 20
NUM_SUPERS = L // LPS

_mesh = plsc.VectorSubcoreMesh(core_axis_name="c", subcore_axis_name="s")


@functools.partial(
    pl.kernel,
    mesh=_mesh,
    out_type=jax.ShapeDtypeStruct((L, B // 2, 2 * D), jnp.float32),
    scratch_types=[
        pltpu.VMEM((LPS, 2, FW), jnp.int32),
        pltpu.VMEM((LPS, 2, FW), jnp.int32),
        pltpu.VMEM((2, FW, D), jnp.float32),
        pltpu.VMEM((2, FW, D), jnp.float32),
        pltpu.SemaphoreType.DMA,
        pltpu.SemaphoreType.DMA,
        pltpu.SemaphoreType.DMA,
        pltpu.SemaphoreType.DMA,
    ],
    compiler_params=pltpu.CompilerParams(use_tc_tiling_on_sc=False),
)
def _emb_lookup_fused(idx_hbm, table_hbm, out_hbm,
                      idx_v0, idx_v1, rows0, rows1, sg0, sg1, ss0, ss1):
    wid = lax.axis_index("s") * NUM_CORES + lax.axis_index("c")
    wf = wid * FW
    idxb = (idx_v0, idx_v1)
    rows = (rows0, rows1)
    sg = (sg0, sg1)
    ss = (ss0, ss1)

    def gather_start(sb, li_local, b):
        for eo in range(2):
            pltpu.async_copy(
                table_hbm.at[idxb[sb].at[li_local, eo]],
                rows[b].at[eo], sg[b])

    def gather_wait(b):
        for eo in range(2):
            pltpu.make_async_copy(
                table_hbm.at[idx_v0.at[0, 0]], rows[b].at[eo], sg[b]).wait()

    def store_start(l, b):
        for eo in range(2):
            pltpu.async_copy(
                rows[b].at[eo],
                out_hbm.at[l, pl.ds(wf, FW), pl.ds(eo * D, D)], ss[b])

    def store_wait(b):
        for eo in range(2):
            pltpu.make_async_copy(
                rows[b].at[eo],
                out_hbm.at[0, pl.ds(wf, FW), pl.ds(eo * D, D)], ss[b]).wait()

    pltpu.sync_copy(
        idx_hbm.at[pl.ds(0, LPS), :, pl.ds(wf, FW)], idx_v0)
    gather_start(0, 0, 1)
    store_start(0, 0)

    def super_pair(sp, _):
        for sb in (0, 1):
            s = 2 * sp + sb
            pltpu.sync_copy(
                idx_hbm.at[pl.ds(s * LPS, LPS), :, pl.ds(wf, FW)], idxb[sb])

            def pos_pair(p, _):
                for b in (0, 1):
                    li = 2 * p + b
                    l = s * LPS + li
                    store_wait(b)
                    gather_start(sb, li, b)
                    gather_wait(1 - b)
                    store_start(jnp.maximum(l - 1, 0), 1 - b)
                return 0

            lax.fori_loop(0, LPS // 2, pos_pair, 0)
        return 0

    lax.fori_loop(0, NUM_SUPERS // 2, super_pair, 0)

    gather_wait(1)
    store_start(L - 1, 1)
    store_wait(0)
    store_wait(1)


def kernel(indices, table):
    idx_eo = indices.T.reshape(L, B // 2, 2).transpose(0, 2, 1)
    out_f = _emb_lookup_fused(idx_eo, table)
    return out_f.reshape(L, B, D).transpose(1, 0, 2)
